# baseline (speedup 1.0000x reference)
import functools

import jax
import jax.numpy as jnp
from jax.experimental import pallas as pl
from jax.experimental.pallas import tpu as pltpu


def _gat_kernel(x_ref, w_ref, as_ref, ad_ref, mask_ref, bias_ref, out_ref,
                *, n, t_steps, heads, dim):
    m = mask_ref[...]
    rows = jax.lax.broadcasted_iota(jnp.int32, (n, n), 0)
    cols = jax.lax.broadcasted_iota(jnp.int32, (n, n), 1)
    maskadd = jnp.where((m != 0) | (rows == cols), 0.0,
                        -1e30).astype(jnp.bfloat16)
    w = w_ref[...]
    b = bias_ref[...]
    inv_h = jnp.float32(1.0 / heads)
    ones_col = jnp.ones((n, 1), dtype=jnp.bfloat16)
    slope = jnp.bfloat16(0.2)
    for t in range(t_steps):
        ht = jnp.dot(x_ref[:, t, :], w,
                     preferred_element_type=jnp.float32)
        a_src = jnp.dot(ht, as_ref[...],
                        preferred_element_type=jnp.float32
                        ).astype(jnp.bfloat16)
        a_dst = jax.lax.dot_general(
            ad_ref[...], ht, (((1,), (1,)), ((), ())),
            preferred_element_type=jnp.float32
            ).astype(jnp.bfloat16)
        ht_bf = ht.astype(jnp.bfloat16)
        acc = None
        for hh in range(heads):
            s = a_src[:, hh:hh + 1] + a_dst[hh:hh + 1, :] + maskadd
            s = jnp.maximum(s, slope * s)
            ex = jnp.exp2(s)
            lhs = jnp.concatenate(
                [ht_bf[:, hh * dim:(hh + 1) * dim], ones_col], axis=1)
            o_aug = jax.lax.dot_general(
                lhs, ex, (((0,), (0,)), ((), ())),
                preferred_element_type=jnp.float32)
            o = o_aug[:dim, :] / (o_aug[dim:, :] + 1e-16)
            acc = o if acc is None else acc + o
        out_ref[:, t * dim:(t + 1) * dim] = jnp.transpose(acc * inv_h) + b


def kernel(x, node_matrix, W, att_src, att_dst, bias):
    n, t_steps, in_dim = x.shape
    heads, dim = att_src.shape[1], att_src.shape[2]
    hc = heads * dim

    eye = jnp.eye(heads, dtype=jnp.float32)
    log2e = jnp.float32(1.4426950408889634)
    as_bd = (att_src.reshape(heads, dim)[:, :, None]
             * eye[:, None, :]).reshape(hc, heads) * log2e
    ad_bd = (att_dst.reshape(heads, dim)[:, None, :]
             * eye[:, :, None]).reshape(heads, hc) * log2e
    bias_row = bias.reshape(1, dim).astype(jnp.float32)

    body = functools.partial(_gat_kernel, n=n, t_steps=t_steps,
                             heads=heads, dim=dim)
    out = pl.pallas_call(
        body,
        out_shape=jax.ShapeDtypeStruct((n, t_steps * dim), jnp.float32),
    )(x.astype(jnp.float32), W, as_bd, ad_bd, node_matrix, bias_row)
    return out.reshape(n, t_steps, dim)


# speedup vs baseline: 1.1560x; 1.1560x over previous
import functools

import jax
import jax.numpy as jnp
from jax.experimental import pallas as pl
from jax.experimental.pallas import tpu as pltpu


def _gat_kernel(x_ref, w_ref, as_ref, ad_ref, mask_hbm_ref, bias_ref,
                out_ref, mask_scr, dma_sem, *, n, t_steps, heads, dim):
    copy = pltpu.make_async_copy(mask_hbm_ref, mask_scr, dma_sem)
    copy.start()
    w = w_ref[...]
    b = bias_ref[...]
    inv_h = jnp.float32(1.0 / heads)
    ones_col = jnp.ones((n, 1), dtype=jnp.bfloat16)
    slope = jnp.bfloat16(0.2)
    hts, asrcs, adsts = [], [], []
    for t in range(t_steps):
        ht = jnp.dot(x_ref[:, t, :], w,
                     preferred_element_type=jnp.float32)
        asrcs.append(jnp.dot(ht, as_ref[...],
                             preferred_element_type=jnp.float32
                             ).astype(jnp.bfloat16))
        adsts.append(jax.lax.dot_general(
            ad_ref[...], ht, (((1,), (1,)), ((), ())),
            preferred_element_type=jnp.float32
            ).astype(jnp.bfloat16))
        hts.append(ht.astype(jnp.bfloat16))
    copy.wait()
    m = mask_scr[...]
    rows = jax.lax.broadcasted_iota(jnp.int32, (n, n), 0)
    cols = jax.lax.broadcasted_iota(jnp.int32, (n, n), 1)
    maskadd = jnp.where((m != 0) | (rows == cols), 0.0,
                        -1e30).astype(jnp.bfloat16)
    for t in range(t_steps):
        a_src, a_dst, ht_bf = asrcs[t], adsts[t], hts[t]
        acc = None
        for hh in range(heads):
            s = a_src[:, hh:hh + 1] + a_dst[hh:hh + 1, :] + maskadd
            s = jnp.maximum(s, slope * s)
            ex = jnp.exp2(s)
            lhs = jnp.concatenate(
                [ht_bf[:, hh * dim:(hh + 1) * dim], ones_col], axis=1)
            o_aug = jax.lax.dot_general(
                lhs, ex, (((0,), (0,)), ((), ())),
                preferred_element_type=jnp.float32)
            o = o_aug[:dim, :] / (o_aug[dim:, :] + 1e-16)
            acc = o if acc is None else acc + o
        out_ref[t * dim:(t + 1) * dim, :] = acc * inv_h + b.reshape(dim, 1)


def kernel(x, node_matrix, W, att_src, att_dst, bias):
    n, t_steps, in_dim = x.shape
    heads, dim = att_src.shape[1], att_src.shape[2]
    hc = heads * dim

    eye = jnp.eye(heads, dtype=jnp.float32)
    log2e = jnp.float32(1.4426950408889634)
    as_bd = (att_src.reshape(heads, dim)[:, :, None]
             * eye[:, None, :]).reshape(hc, heads) * log2e
    ad_bd = (att_dst.reshape(heads, dim)[:, None, :]
             * eye[:, :, None]).reshape(heads, hc) * log2e
    bias_row = bias.reshape(1, dim).astype(jnp.float32)

    body = functools.partial(_gat_kernel, n=n, t_steps=t_steps,
                             heads=heads, dim=dim)
    out_t = pl.pallas_call(
        body,
        in_specs=[
            pl.BlockSpec((n, t_steps, in_dim), lambda: (0, 0, 0)),
            pl.BlockSpec((in_dim, hc), lambda: (0, 0)),
            pl.BlockSpec((hc, heads), lambda: (0, 0)),
            pl.BlockSpec((heads, hc), lambda: (0, 0)),
            pl.BlockSpec(memory_space=pl.ANY),
            pl.BlockSpec((1, dim), lambda: (0, 0)),
        ],
        out_specs=pl.BlockSpec((t_steps * dim, n), lambda: (0, 0)),
        out_shape=jax.ShapeDtypeStruct((t_steps * dim, n), jnp.float32),
        scratch_shapes=[
            pltpu.VMEM((n, n), jnp.int32),
            pltpu.SemaphoreType.DMA,
        ],
    )(x.astype(jnp.float32), W, as_bd, ad_bd, node_matrix, bias_row)
    return jnp.transpose(out_t.reshape(t_steps, dim, n), (2, 0, 1))
